# baseline (speedup 1.0000x reference)
import functools

import jax
import jax.numpy as jnp
from jax.experimental import pallas as pl
from jax.experimental.pallas import tpu as pltpu


def _body(targ_ref, x_ref, ll_ref, loss_ref, lams_ref, cnt_ref, s_ref, nll_ref,
          *, b_total):
    i = pl.program_id(0)
    nb = pl.num_programs(0)
    x = x_ref[...]
    t = targ_ref[0, 0, :]
    bb, c = x.shape

    m = jnp.max(x, axis=1, keepdims=True)
    ex = jnp.exp(x - m)
    se = jnp.sum(ex, axis=1, keepdims=True)
    lse = m + jnp.log(se)
    neg_logp = lse - x
    probs = ex / se

    cidx = jax.lax.broadcasted_iota(jnp.int32, (bb, c), 1)
    top1 = jnp.min(jnp.where(x == m, cidx, c), axis=1)
    corr = (t == top1).astype(jnp.float32)[:, None]
    onehot = (cidx == t[:, None]).astype(jnp.float32)

    e_blk = probs * corr
    lam_blk = jax.lax.dot_general(
        onehot, e_blk, (((0,), (0,)), ((), ())),
        preferred_element_type=jnp.float32, precision=jax.lax.Precision.HIGHEST)
    s_blk = jax.lax.dot_general(
        onehot, neg_logp, (((0,), (0,)), ((), ())),
        preferred_element_type=jnp.float32, precision=jax.lax.Precision.HIGHEST)
    cnt_blk = jnp.sum(onehot * corr, axis=0)[None, :]
    nll_blk = jnp.sum(onehot * neg_logp)

    @pl.when(i == 0)
    def _init():
        lams_ref[...] = lam_blk
        s_ref[...] = s_blk
        cnt_ref[...] = cnt_blk
        nll_ref[0, 0] = nll_blk

    @pl.when(i > 0)
    def _acc():
        lams_ref[...] += lam_blk
        s_ref[...] += s_blk
        cnt_ref[...] += cnt_blk
        nll_ref[0, 0] += nll_blk

    @pl.when(i == nb - 1)
    def _fin():
        soft_sum = jnp.sum(s_ref[...] * ll_ref[...])
        val = 0.5 * (nll_ref[0, 0] + soft_sum) / b_total
        loss_ref[...] = jnp.full((1, 1), val, jnp.float32)


def kernel(output, target, loss_lams):
    bn, cn = output.shape
    bb = 512
    nb = bn // bb
    targ3 = target.reshape(nb, 1, bb)

    loss, lams, cnt = pl.pallas_call(
        functools.partial(_body, b_total=bn),
        grid=(nb,),
        in_specs=[
            pl.BlockSpec((1, 1, bb), lambda i: (i, 0, 0)),
            pl.BlockSpec((bb, cn), lambda i: (i, 0)),
            pl.BlockSpec((cn, cn), lambda i: (0, 0)),
        ],
        out_specs=[
            pl.BlockSpec((1, 1), lambda i: (0, 0)),
            pl.BlockSpec((cn, cn), lambda i: (0, 0)),
            pl.BlockSpec((1, cn), lambda i: (0, 0)),
        ],
        out_shape=[
            jax.ShapeDtypeStruct((1, 1), jnp.float32),
            jax.ShapeDtypeStruct((cn, cn), jnp.float32),
            jax.ShapeDtypeStruct((1, cn), jnp.float32),
        ],
        scratch_shapes=[
            pltpu.VMEM((cn, cn), jnp.float32),
            pltpu.SMEM((1, 1), jnp.float32),
        ],
    )(targ3, output, loss_lams)

    return loss[0, 0], lams, cnt[0]


# speedup vs baseline: 2.6036x; 2.6036x over previous
import functools

import jax
import jax.numpy as jnp
from jax.experimental import pallas as pl
from jax.experimental.pallas import tpu as pltpu


def _body(targ_ref, x_ref, ll_ref, loss_ref, lams_ref, cnt_ref, xs_ref,
          *, b_total):
    i = pl.program_id(0)
    nb = pl.num_programs(0)
    x = x_ref[...]
    t = targ_ref[0, 0, :]
    bb, c = x.shape

    m = jnp.max(x, axis=1, keepdims=True)
    ex = jnp.exp(x - m)
    se = jnp.sum(ex, axis=1, keepdims=True)
    lse = m + jnp.log(se)

    cidx = jax.lax.broadcasted_iota(jnp.int32, (bb, c), 1)
    top1 = jnp.min(jnp.where(x == m, cidx, c), axis=1)
    corr = (t == top1).astype(jnp.float32)[:, None]
    oh16 = (cidx == t[:, None]).astype(jnp.bfloat16)

    e16 = (ex * (corr / se)).astype(jnp.bfloat16)
    l16 = (lse - x).astype(jnp.bfloat16)
    lam_blk = jax.lax.dot_general(
        oh16, e16, (((0,), (0,)), ((), ())),
        preferred_element_type=jnp.float32)
    s_blk = jax.lax.dot_general(
        oh16, l16, (((0,), (0,)), ((), ())),
        preferred_element_type=jnp.float32)

    @pl.when(i == 0)
    def _init():
        lams_ref[...] = lam_blk
        xs_ref[...] = s_blk

    @pl.when(i > 0)
    def _acc():
        lams_ref[...] += lam_blk
        xs_ref[...] += s_blk

    @pl.when(i == nb - 1)
    def _fin():
        s_tab = xs_ref[...]
        soft_sum = jnp.sum(s_tab * ll_ref[...])
        ridx = jax.lax.broadcasted_iota(jnp.int32, (c, c), 0)
        kidx = jax.lax.broadcasted_iota(jnp.int32, (c, c), 1)
        nll_sum = jnp.sum(jnp.where(ridx == kidx, s_tab, 0.0))
        cnt_ref[...] = jnp.sum(lams_ref[...], axis=1)[None, :]
        val = 0.5 * (nll_sum + soft_sum) / b_total
        loss_ref[...] = jnp.full((1, 1), val, jnp.float32)


def kernel(output, target, loss_lams):
    bn, cn = output.shape
    bb = 512
    nb = bn // bb
    targ3 = target.reshape(nb, 1, bb)

    loss, lams, cnt = pl.pallas_call(
        functools.partial(_body, b_total=bn),
        grid=(nb,),
        in_specs=[
            pl.BlockSpec((1, 1, bb), lambda i: (i, 0, 0)),
            pl.BlockSpec((bb, cn), lambda i: (i, 0)),
            pl.BlockSpec((cn, cn), lambda i: (0, 0)),
        ],
        out_specs=[
            pl.BlockSpec((1, 1), lambda i: (0, 0)),
            pl.BlockSpec((cn, cn), lambda i: (0, 0)),
            pl.BlockSpec((1, cn), lambda i: (0, 0)),
        ],
        out_shape=[
            jax.ShapeDtypeStruct((1, 1), jnp.float32),
            jax.ShapeDtypeStruct((cn, cn), jnp.float32),
            jax.ShapeDtypeStruct((1, cn), jnp.float32),
        ],
        scratch_shapes=[
            pltpu.VMEM((cn, cn), jnp.float32),
        ],
    )(targ3, output, loss_lams)

    return loss[0, 0], lams, cnt[0]


# speedup vs baseline: 2.9018x; 1.1146x over previous
import functools

import jax
import jax.numpy as jnp
from jax.experimental import pallas as pl
from jax.experimental.pallas import tpu as pltpu


def _body(targ_ref, x_ref, ll_ref, loss_ref, lams_ref, cnt_ref, xs_ref,
          *, b_total):
    i = pl.program_id(0)
    nb = pl.num_programs(0)
    x = x_ref[...]
    t = targ_ref[0, 0, :]
    bb, c = x.shape

    m = jnp.max(x, axis=1, keepdims=True)
    ex = jnp.exp(x - m)
    se = jnp.sum(ex, axis=1, keepdims=True)
    lse = m + jnp.log(se)

    cidx = jax.lax.broadcasted_iota(jnp.int32, (bb, c), 1)
    top1 = jnp.min(jnp.where(x == m, cidx, c), axis=1)
    corr = (t == top1).astype(jnp.float32)[:, None]
    oh16 = (cidx == t[:, None]).astype(jnp.bfloat16)

    e16 = (ex * (corr / se)).astype(jnp.bfloat16)
    l16 = (lse - x).astype(jnp.bfloat16)
    lam_blk = jax.lax.dot_general(
        oh16, e16, (((0,), (0,)), ((), ())),
        preferred_element_type=jnp.float32)
    s_blk = jax.lax.dot_general(
        oh16, l16, (((0,), (0,)), ((), ())),
        preferred_element_type=jnp.float32)

    @pl.when(i == 0)
    def _init():
        lams_ref[...] = lam_blk
        xs_ref[...] = s_blk

    @pl.when(i > 0)
    def _acc():
        lams_ref[...] += lam_blk
        xs_ref[...] += s_blk

    @pl.when(i == nb - 1)
    def _fin():
        s_tab = xs_ref[...]
        soft_sum = jnp.sum(s_tab * ll_ref[...])
        ridx = jax.lax.broadcasted_iota(jnp.int32, (c, c), 0)
        kidx = jax.lax.broadcasted_iota(jnp.int32, (c, c), 1)
        nll_sum = jnp.sum(jnp.where(ridx == kidx, s_tab, 0.0))
        cnt_ref[...] = jnp.sum(lams_ref[...], axis=1)[None, :]
        val = 0.5 * (nll_sum + soft_sum) / b_total
        loss_ref[...] = jnp.full((1, 1), val, jnp.float32)


def kernel(output, target, loss_lams):
    bn, cn = output.shape
    bb = 1024
    nb = bn // bb
    targ3 = target.reshape(nb, 1, bb)

    loss, lams, cnt = pl.pallas_call(
        functools.partial(_body, b_total=bn),
        grid=(nb,),
        in_specs=[
            pl.BlockSpec((1, 1, bb), lambda i: (i, 0, 0)),
            pl.BlockSpec((bb, cn), lambda i: (i, 0)),
            pl.BlockSpec((cn, cn), lambda i: (0, 0)),
        ],
        out_specs=[
            pl.BlockSpec((1, 1), lambda i: (0, 0)),
            pl.BlockSpec((cn, cn), lambda i: (0, 0)),
            pl.BlockSpec((1, cn), lambda i: (0, 0)),
        ],
        out_shape=[
            jax.ShapeDtypeStruct((1, 1), jnp.float32),
            jax.ShapeDtypeStruct((cn, cn), jnp.float32),
            jax.ShapeDtypeStruct((1, cn), jnp.float32),
        ],
        scratch_shapes=[
            pltpu.VMEM((cn, cn), jnp.float32),
        ],
    )(targ3, output, loss_lams)

    return loss[0, 0], lams, cnt[0]


# speedup vs baseline: 3.0089x; 1.0369x over previous
import functools

import jax
import jax.numpy as jnp
from jax.experimental import pallas as pl
from jax.experimental.pallas import tpu as pltpu


def _body(targ_ref, x_ref, ll_ref, loss_ref, lams_ref, cnt_ref, xs_ref,
          *, b_total):
    i = pl.program_id(0)
    nb = pl.num_programs(0)
    x = x_ref[...]
    t = targ_ref[0, 0, :]
    bb, c = x.shape

    m = jnp.max(x, axis=1, keepdims=True)
    ex = jnp.exp(x - m)
    se = jnp.sum(ex, axis=1, keepdims=True)
    lse = m + jnp.log(se)

    cidx = jax.lax.broadcasted_iota(jnp.int32, (bb, c), 1)
    top1 = jnp.min(jnp.where(x == m, cidx, c), axis=1)
    corr = (t == top1).astype(jnp.float32)[:, None]
    oh16 = (cidx == t[:, None]).astype(jnp.bfloat16)

    e16 = (ex * (corr / se)).astype(jnp.bfloat16)
    l16 = (lse - x).astype(jnp.bfloat16)
    lam_blk = jax.lax.dot_general(
        oh16, e16, (((0,), (0,)), ((), ())),
        preferred_element_type=jnp.float32)
    s_blk = jax.lax.dot_general(
        oh16, l16, (((0,), (0,)), ((), ())),
        preferred_element_type=jnp.float32)

    @pl.when(i == 0)
    def _init():
        lams_ref[...] = lam_blk
        xs_ref[...] = s_blk

    @pl.when(i > 0)
    def _acc():
        lams_ref[...] += lam_blk
        xs_ref[...] += s_blk

    @pl.when(i == nb - 1)
    def _fin():
        s_tab = xs_ref[...]
        soft_sum = jnp.sum(s_tab * ll_ref[...])
        ridx = jax.lax.broadcasted_iota(jnp.int32, (c, c), 0)
        kidx = jax.lax.broadcasted_iota(jnp.int32, (c, c), 1)
        nll_sum = jnp.sum(jnp.where(ridx == kidx, s_tab, 0.0))
        cnt_ref[...] = jnp.sum(lams_ref[...], axis=1)[None, :]
        val = 0.5 * (nll_sum + soft_sum) / b_total
        loss_ref[...] = jnp.full((1, 1), val, jnp.float32)


def kernel(output, target, loss_lams):
    bn, cn = output.shape
    bb = 2048
    nb = bn // bb
    targ3 = target.reshape(nb, 1, bb)

    loss, lams, cnt = pl.pallas_call(
        functools.partial(_body, b_total=bn),
        grid=(nb,),
        in_specs=[
            pl.BlockSpec((1, 1, bb), lambda i: (i, 0, 0)),
            pl.BlockSpec((bb, cn), lambda i: (i, 0)),
            pl.BlockSpec((cn, cn), lambda i: (0, 0)),
        ],
        out_specs=[
            pl.BlockSpec((1, 1), lambda i: (0, 0)),
            pl.BlockSpec((cn, cn), lambda i: (0, 0)),
            pl.BlockSpec((1, cn), lambda i: (0, 0)),
        ],
        out_shape=[
            jax.ShapeDtypeStruct((1, 1), jnp.float32),
            jax.ShapeDtypeStruct((cn, cn), jnp.float32),
            jax.ShapeDtypeStruct((1, cn), jnp.float32),
        ],
        scratch_shapes=[
            pltpu.VMEM((cn, cn), jnp.float32),
        ],
    )(targ3, output, loss_lams)

    return loss[0, 0], lams, cnt[0]
